# baseline (speedup 1.0000x reference)
import functools

import jax
import jax.numpy as jnp
from jax import lax
from jax.experimental import pallas as pl
from jax.experimental.pallas import tpu as pltpu
from jax.experimental.pallas import tpu_sc as plsc

B, SIZE, VOCAB, EMBED, HIDDEN = 1024, 50, 100000, 64, 512
NIDX = B * SIZE

NC, NS = 2, 16
NW = NC * NS
ROWS_PER_W = NIDX // NW

VTILE = 2048
TPC = 24
RAGGED_T = 48
K_STREAMS = 4
SUBW = VTILE // K_STREAMS


def _sc_gather(table, idx):
    mesh = plsc.VectorSubcoreMesh(core_axis_name="c", subcore_axis_name="s")

    @functools.partial(
        pl.kernel,
        out_type=jax.ShapeDtypeStruct((NIDX, EMBED), jnp.float32),
        mesh=mesh,
        scratch_types=[
            pltpu.VMEM((ROWS_PER_W,), jnp.int32),
            pltpu.VMEM((ROWS_PER_W, EMBED), jnp.float32),
            pltpu.SemaphoreType.DMA,
        ],
        compiler_params=pltpu.CompilerParams(use_tc_tiling_on_sc=False),
    )
    def gather_kernel(table_hbm, idx_hbm, out_hbm, idx_v, rows_v, sem):
        wid = lax.axis_index("s") * NC + lax.axis_index("c")
        base = wid * ROWS_PER_W
        pltpu.sync_copy(idx_hbm.at[pl.ds(base, ROWS_PER_W)], idx_v)
        pltpu.async_copy(table_hbm.at[idx_v], rows_v, sem).wait()
        pltpu.sync_copy(rows_v, out_hbm.at[pl.ds(base, ROWS_PER_W)])

    return gather_kernel(table, idx)


def _mm1_body(flat_ref, w1_ref, b1_ref, h_ref):
    acc = jnp.dot(
        flat_ref[...].astype(jnp.bfloat16),
        w1_ref[...].astype(jnp.bfloat16),
        preferred_element_type=jnp.float32,
    )
    h_ref[...] = jnp.maximum(acc + b1_ref[...], 0.0).astype(jnp.bfloat16)


def _mm2_body(h_ref, b2_ref, w2_hbm, out_hbm, w2_buf, out_buf, in_sems, out_sems):
    c = pl.program_id(0)
    j = pl.program_id(1)
    t = c * TPC + j

    def in_copy(tt, k, width):
        col = tt * VTILE + k * SUBW
        return pltpu.make_async_copy(
            w2_hbm.at[:, pl.ds(col, width)],
            w2_buf.at[lax.rem(tt, 3), :, pl.ds(k * SUBW, width)],
            in_sems.at[lax.rem(tt, 3), k],
        )

    def start_in(tt):
        for k in range(K_STREAMS):
            in_copy(tt, k, SUBW).start()

    def out_copy(oslot, tt, k, width):
        col = tt * VTILE + k * SUBW
        return pltpu.make_async_copy(
            out_buf.at[oslot, :, pl.ds(k * SUBW, width)],
            out_hbm.at[:, pl.ds(col, width)],
            out_sems.at[oslot, k],
        )

    @pl.when(j == 0)
    def _():
        start_in(t)
        start_in(t + 1)

    @pl.when(j < TPC - 2)
    def _():
        start_in(t + 2)

    for k in range(K_STREAMS):
        in_copy(t, k, SUBW).wait()

    @pl.when(j >= 2)
    def _():
        for k in range(K_STREAMS):
            out_copy(lax.rem(j, 2), t - 2, k, SUBW).wait()

    w2v = w2_buf[lax.rem(t, 3)].astype(jnp.bfloat16)
    acc = jnp.dot(h_ref[...], w2v, preferred_element_type=jnp.float32)
    out_buf[lax.rem(j, 2)] = acc + b2_ref[...]

    for k in range(K_STREAMS):
        out_copy(lax.rem(j, 2), t, k, SUBW).start()

    @pl.when(j == TPC - 1)
    def _():
        for k in range(K_STREAMS):
            out_copy(0, t - 1, k, SUBW).wait()
        for k in range(K_STREAMS):
            out_copy(1, t, k, SUBW).wait()


def _mm2_tail_body(h_ref, w2_ref, b2_ref, prev_ref, out_ref):
    del prev_ref
    acc = jnp.dot(
        h_ref[...],
        w2_ref[...].astype(jnp.bfloat16),
        preferred_element_type=jnp.float32,
    )
    out_ref[...] = acc + b2_ref[...]


def kernel(x, embed, W1, b1, W2, b2):
    idx = x.reshape(-1).astype(jnp.int32)
    flat_rows = _sc_gather(embed, idx)
    flat = flat_rows.reshape(B, SIZE * EMBED)

    b1_2d = b1.reshape(1, HIDDEN)
    b2_2d = b2.reshape(1, VOCAB)

    h = pl.pallas_call(
        _mm1_body,
        grid=(2,),
        in_specs=[
            pl.BlockSpec((B // 2, SIZE * EMBED), lambda i: (i, 0)),
            pl.BlockSpec((SIZE * EMBED, HIDDEN), lambda i: (0, 0)),
            pl.BlockSpec((1, HIDDEN), lambda i: (0, 0)),
        ],
        out_specs=pl.BlockSpec((B // 2, HIDDEN), lambda i: (i, 0)),
        out_shape=jax.ShapeDtypeStruct((B, HIDDEN), jnp.bfloat16),
        compiler_params=pltpu.CompilerParams(
            dimension_semantics=("parallel",),
        ),
    )(flat, W1, b1_2d)

    out = pl.pallas_call(
        _mm2_body,
        grid=(2, TPC),
        in_specs=[
            pl.BlockSpec((B, HIDDEN), lambda c, j: (0, 0)),
            pl.BlockSpec((1, VTILE), lambda c, j: (0, c * TPC + j)),
            pl.BlockSpec(memory_space=pl.ANY),
        ],
        out_specs=pl.BlockSpec(memory_space=pl.ANY),
        out_shape=jax.ShapeDtypeStruct((B, VOCAB), jnp.float32),
        scratch_shapes=[
            pltpu.VMEM((3, HIDDEN, VTILE), jnp.float32),
            pltpu.VMEM((2, B, VTILE), jnp.float32),
            pltpu.SemaphoreType.DMA((3, K_STREAMS)),
            pltpu.SemaphoreType.DMA((2, K_STREAMS)),
        ],
        compiler_params=pltpu.CompilerParams(
            dimension_semantics=("parallel", "arbitrary"),
            vmem_limit_bytes=60 * 1024 * 1024,
        ),
    )(h, b2_2d, W2)

    out = pl.pallas_call(
        _mm2_tail_body,
        grid=(1,),
        in_specs=[
            pl.BlockSpec((B, HIDDEN), lambda i: (0, 0)),
            pl.BlockSpec((HIDDEN, VTILE), lambda i: (0, RAGGED_T)),
            pl.BlockSpec((1, VTILE), lambda i: (0, RAGGED_T)),
            pl.BlockSpec(memory_space=pl.ANY),
        ],
        out_specs=pl.BlockSpec((B, VTILE), lambda i: (0, RAGGED_T)),
        out_shape=jax.ShapeDtypeStruct((B, VOCAB), jnp.float32),
        input_output_aliases={3: 0},
        compiler_params=pltpu.CompilerParams(
            dimension_semantics=("arbitrary",),
        ),
    )(h, W2, b2_2d, out)
    return out
